# baseline (speedup 1.0000x reference)
import functools

import jax
import jax.numpy as jnp
from jax import lax
from jax.experimental import pallas as pl
from jax.experimental.pallas import tpu as pltpu
from jax.experimental.pallas import tpu_sc as plsc

D_MODEL = 768
D_FF = 3072
N_EXPERTS = 8
TOP_K = 2
T_TOKENS = 2048
CAP = 768
TM = 256
NW = 32
LANES = 16


def _worker_id():
    return lax.axis_index("s") * 2 + lax.axis_index("c")


def _make_dispatch():
    mesh = plsc.VectorSubcoreMesh(core_axis_name="c", subcore_axis_name="s")
    A = TOP_K * T_TOKENS
    APW = A // NW

    @functools.partial(
        pl.kernel,
        out_type=jax.ShapeDtypeStruct((N_EXPERTS * CAP, D_MODEL), jnp.float32),
        mesh=mesh,
        scratch_types=[
            pltpu.VMEM((APW,), jnp.int32),
            pltpu.VMEM((APW, D_MODEL), jnp.float32),
            pltpu.SemaphoreType.DMA,
        ],
    )
    def disp(x_hbm, dest_hbm, xp_hbm, idx_v, rows_v, sem):
        base = _worker_id() * APW
        tbase = lax.rem(base, T_TOKENS)
        pltpu.sync_copy(dest_hbm.at[pl.ds(base, APW)], idx_v)
        pltpu.sync_copy(x_hbm.at[pl.ds(tbase, APW)], rows_v)
        pltpu.async_copy(rows_v, xp_hbm.at[idx_v], sem).wait()

    return disp


_make_dispatch = functools.cache(_make_dispatch)


def _make_combine():
    mesh = plsc.VectorSubcoreMesh(core_axis_name="c", subcore_axis_name="s")
    TPW = T_TOKENS // NW

    @functools.partial(
        pl.kernel,
        out_type=jax.ShapeDtypeStruct((T_TOKENS, D_MODEL), jnp.float32),
        mesh=mesh,
        scratch_types=[
            pltpu.VMEM((TPW,), jnp.int32),
            pltpu.VMEM((TPW,), jnp.int32),
            pltpu.VMEM((TPW, 128), jnp.float32),
            pltpu.VMEM((TPW, 128), jnp.float32),
            pltpu.VMEM((TPW, D_MODEL), jnp.float32),
            pltpu.VMEM((TPW, D_MODEL), jnp.float32),
            pltpu.SemaphoreType.DMA,
        ],
    )
    def comb(eo_hbm, d0_hbm, d1_hbm, w0_hbm, w1_hbm, out_hbm,
             i0_v, i1_v, w0_v, w1_v, r0_v, r1_v, sem):
        base = _worker_id() * TPW
        pltpu.sync_copy(d0_hbm.at[pl.ds(base, TPW)], i0_v)
        pltpu.sync_copy(d1_hbm.at[pl.ds(base, TPW)], i1_v)
        pltpu.sync_copy(w0_hbm.at[pl.ds(base, TPW)], w0_v)
        pltpu.sync_copy(w1_hbm.at[pl.ds(base, TPW)], w1_v)
        pltpu.async_copy(eo_hbm.at[i0_v], r0_v, sem).wait()
        pltpu.async_copy(eo_hbm.at[i1_v], r1_v, sem).wait()

        def body(t, carry):
            bw0 = w0_v[t, pl.ds(0, LANES)]
            bw1 = w1_v[t, pl.ds(0, LANES)]
            for j in range(D_MODEL // LANES):
                sl = pl.ds(j * LANES, LANES)
                r0_v[t, sl] = bw0 * r0_v[t, sl] + bw1 * r1_v[t, sl]
            return carry

        lax.fori_loop(0, TPW, body, 0)
        pltpu.sync_copy(r0_v, out_hbm.at[pl.ds(base, TPW)])

    return comb


_make_combine = functools.cache(_make_combine)


def _router_kernel(x_ref, rwt_ref, w0_ref, w1_ref, i0_ref, i1_ref, ps_ref):
    T = T_TOKENS
    xl = x_ref[...]
    logits = jnp.dot(xl.astype(jnp.bfloat16), rwt_ref[...].astype(jnp.bfloat16),
                     preferred_element_type=jnp.float32)
    lane = lax.broadcasted_iota(jnp.int32, (T, 128), 1)
    ml = jnp.where(lane < N_EXPERTS, logits, -jnp.inf)
    mx = jnp.max(ml, axis=1, keepdims=True)
    ex = jnp.exp(ml - mx)
    probs = ex / jnp.sum(ex, axis=1, keepdims=True)
    m0 = jnp.max(probs, axis=1, keepdims=True)
    i0 = jnp.min(jnp.where(probs == m0, lane, 128), axis=1, keepdims=True)
    pr1 = jnp.where(lane == i0, -1.0, probs)
    m1 = jnp.max(pr1, axis=1, keepdims=True)
    i1 = jnp.min(jnp.where(pr1 == m1, lane, 128), axis=1, keepdims=True)
    den = m0 + m1 + 1e-8
    w0_ref[...] = jnp.broadcast_to(m0 / den, (T, 128))
    w1_ref[...] = jnp.broadcast_to(m1 / den, (T, 128))
    i0_ref[...] = jnp.broadcast_to(i0, (T, 128))
    i1_ref[...] = jnp.broadcast_to(i1, (T, 128))
    ps_ref[...] = jnp.broadcast_to(jnp.sum(probs, axis=0, keepdims=True), (8, 128))


def _router(xf, router_w):
    T = T_TOKENS
    rwt = jnp.pad(router_w, ((0, 128 - N_EXPERTS), (0, 0))).T
    return pl.pallas_call(
        _router_kernel,
        out_shape=(
            jax.ShapeDtypeStruct((T, 128), jnp.float32),
            jax.ShapeDtypeStruct((T, 128), jnp.float32),
            jax.ShapeDtypeStruct((T, 128), jnp.int32),
            jax.ShapeDtypeStruct((T, 128), jnp.int32),
            jax.ShapeDtypeStruct((8, 128), jnp.float32),
        ),
    )(xf, rwt)


def _tree_sum(parts):
    while len(parts) > 1:
        nxt = [parts[i] + parts[i + 1] for i in range(0, len(parts) - 1, 2)]
        if len(parts) % 2:
            nxt.append(parts[-1])
        parts = nxt
    return parts[0]


def _ffn_kernel(counts_s, xp_ref, w1c0, w1c1, w1c2, w2c0, w2c1, w2c2,
                eo_ref, w1q, w2q):
    e = pl.program_id(0)
    m = pl.program_id(1)

    @pl.when(m == 0)
    def _quantize_weights():
        def ternarize(chunks, q_ref, crows, nsub):
            sz = crows // nsub
            parts = []
            for ch in chunks:
                for i in range(nsub):
                    parts.append(jnp.sum(jnp.abs(ch[0, 0, pl.ds(i * sz, sz), :])))
            s = 1.0 / jnp.clip(_tree_sum(parts) / (D_MODEL * D_FF), 1e-5, None)
            d = 1.0 / s
            t = 0.5 * d
            for ci, ch in enumerate(chunks):
                for i in range(nsub):
                    w = ch[0, 0, pl.ds(i * sz, sz), :]
                    q_ref[pl.ds(ci * crows + i * sz, sz), :] = jnp.where(
                        w > t, d, jnp.where(w < -t, -d, 0.0)).astype(jnp.bfloat16)

        ternarize((w1c0, w1c1, w1c2), w1q, D_MODEL // 3, 2)
        ternarize((w2c0, w2c1, w2c2), w2q, D_FF // 3, 4)

    @pl.when(m * TM < counts_s[e])
    def _compute():
        xt = xp_ref[0]
        ax = jnp.clip(jnp.max(jnp.abs(xt), axis=-1, keepdims=True), 1e-5, None)
        sx = 127.0 / ax
        rx = ax * (1.0 / 127.0)
        xq = jnp.clip(jnp.round(xt * sx), -128.0, 127.0) * rx
        h = jnp.dot(xq.astype(jnp.bfloat16), w1q[...],
                    preferred_element_type=jnp.float32)
        h = jax.nn.gelu(h)
        ah = jnp.clip(jnp.max(jnp.abs(h), axis=-1, keepdims=True), 1e-5, None)
        sh = 127.0 / ah
        rh = ah * (1.0 / 127.0)
        hq = jnp.clip(jnp.round(h * sh), -128.0, 127.0) * rh
        o = jnp.dot(hq.astype(jnp.bfloat16), w2q[...],
                    preferred_element_type=jnp.float32)
        eo_ref[0] = o


def _row_tile_map(e, m, c):
    last = jnp.maximum(lax.div(c[e] + TM - 1, TM) - 1, 0)
    return (e, jnp.minimum(m, last), 0)


def _chunk_map(j):
    def f(e, m, c):
        return (jnp.where(m > j, jnp.minimum(e + 1, N_EXPERTS - 1), e), j, 0, 0)
    return f


def _ffn(xp, w1, w2, counts):
    grid = (N_EXPERTS, CAP // TM)
    w1r = w1.reshape(N_EXPERTS, 3, D_MODEL // 3, D_FF)
    w2r = w2.reshape(N_EXPERTS, 3, D_FF // 3, D_MODEL)
    return pl.pallas_call(
        _ffn_kernel,
        grid_spec=pltpu.PrefetchScalarGridSpec(
            num_scalar_prefetch=1,
            grid=grid,
            in_specs=[
                pl.BlockSpec((1, TM, D_MODEL), _row_tile_map),
                pl.BlockSpec((1, 1, D_MODEL // 3, D_FF), _chunk_map(0)),
                pl.BlockSpec((1, 1, D_MODEL // 3, D_FF), _chunk_map(1)),
                pl.BlockSpec((1, 1, D_MODEL // 3, D_FF), _chunk_map(2)),
                pl.BlockSpec((1, 1, D_FF // 3, D_MODEL), _chunk_map(0)),
                pl.BlockSpec((1, 1, D_FF // 3, D_MODEL), _chunk_map(1)),
                pl.BlockSpec((1, 1, D_FF // 3, D_MODEL), _chunk_map(2)),
            ],
            out_specs=pl.BlockSpec((1, TM, D_MODEL), _row_tile_map),
            scratch_shapes=[
                pltpu.VMEM((D_MODEL, D_FF), jnp.bfloat16),
                pltpu.VMEM((D_FF, D_MODEL), jnp.bfloat16),
            ],
        ),
        out_shape=jax.ShapeDtypeStruct((N_EXPERTS, CAP, D_MODEL), jnp.float32),
    )(counts, xp, w1r, w1r, w1r, w2r, w2r, w2r)


def kernel(x, router_w, w1, w2):
    B, T, D = x.shape
    xf = x.reshape(T, D)

    w0b, w1b, i0b, i1b, psb = _router(xf, router_w)
    idx_all = jnp.concatenate([i0b[:, 0], i1b[:, 0]])
    oh = jax.nn.one_hot(idx_all, N_EXPERTS, dtype=jnp.int32)
    rank = jnp.cumsum(oh, axis=0) - oh
    local = jnp.minimum(jnp.sum(rank * oh, axis=1), CAP - 1)
    dest = idx_all * CAP + local
    counts = jnp.sum(oh, axis=0).astype(jnp.int32)

    xp = _make_dispatch()(xf, dest)
    eo = _ffn(xp.reshape(N_EXPERTS, CAP, D_MODEL), w1, w2, counts)
    out = _make_combine()(eo.reshape(N_EXPERTS * CAP, D_MODEL),
                          dest[:T], dest[T:], w0b, w1b)

    f = counts.astype(jnp.float32) / (T * TOP_K)
    p = psb[0, :N_EXPERTS] / T
    aux_loss = N_EXPERTS * jnp.sum(f * p)
    return out.reshape(B, T, D), aux_loss


# speedup vs baseline: 1.0286x; 1.0286x over previous
import functools

import jax
import jax.numpy as jnp
from jax import lax
from jax.experimental import pallas as pl
from jax.experimental.pallas import tpu as pltpu
from jax.experimental.pallas import tpu_sc as plsc

D_MODEL = 768
D_FF = 3072
N_EXPERTS = 8
TOP_K = 2
T_TOKENS = 2048
CAP = 768
TM = 256
NW = 32
LANES = 16


def _worker_id():
    return lax.axis_index("s") * 2 + lax.axis_index("c")


def _make_dispatch():
    mesh = plsc.VectorSubcoreMesh(core_axis_name="c", subcore_axis_name="s")
    A = TOP_K * T_TOKENS
    APW = A // NW

    @functools.partial(
        pl.kernel,
        out_type=jax.ShapeDtypeStruct((N_EXPERTS * CAP, D_MODEL), jnp.float32),
        mesh=mesh,
        scratch_types=[
            pltpu.VMEM((APW,), jnp.int32),
            pltpu.VMEM((APW, D_MODEL), jnp.float32),
            pltpu.SemaphoreType.DMA,
        ],
    )
    def disp(x_hbm, dest_hbm, xp_hbm, idx_v, rows_v, sem):
        base = _worker_id() * APW
        tbase = lax.rem(base, T_TOKENS)
        pltpu.sync_copy(dest_hbm.at[pl.ds(base, APW)], idx_v)
        pltpu.sync_copy(x_hbm.at[pl.ds(tbase, APW)], rows_v)
        pltpu.async_copy(rows_v, xp_hbm.at[idx_v], sem).wait()

    return disp


_make_dispatch = functools.cache(_make_dispatch)


def _make_combine():
    mesh = plsc.VectorSubcoreMesh(core_axis_name="c", subcore_axis_name="s")
    TPW = T_TOKENS // NW

    @functools.partial(
        pl.kernel,
        out_type=jax.ShapeDtypeStruct((T_TOKENS, D_MODEL), jnp.float32),
        mesh=mesh,
        scratch_types=[
            pltpu.VMEM((TPW,), jnp.int32),
            pltpu.VMEM((TPW,), jnp.int32),
            pltpu.VMEM((TPW, 128), jnp.float32),
            pltpu.VMEM((TPW, D_MODEL), jnp.float32),
            pltpu.VMEM((TPW, D_MODEL), jnp.float32),
            pltpu.SemaphoreType.DMA,
        ],
    )
    def comb(eo_hbm, d0_hbm, d1_hbm, wv_hbm, out_hbm,
             i0_v, i1_v, wv_v, r0_v, r1_v, sem):
        base = _worker_id() * TPW
        pltpu.sync_copy(d0_hbm.at[pl.ds(base, TPW)], i0_v)
        pltpu.sync_copy(d1_hbm.at[pl.ds(base, TPW)], i1_v)
        pltpu.sync_copy(wv_hbm.at[pl.ds(base, TPW)], wv_v)
        pltpu.async_copy(eo_hbm.at[i0_v], r0_v, sem).wait()
        pltpu.async_copy(eo_hbm.at[i1_v], r1_v, sem).wait()

        def body(t, carry):
            bw0 = wv_v[t, pl.ds(0, LANES)]
            bw1 = wv_v[t, pl.ds(LANES, LANES)]
            for j in range(D_MODEL // LANES):
                sl = pl.ds(j * LANES, LANES)
                r0_v[t, sl] = bw0 * r0_v[t, sl] + bw1 * r1_v[t, sl]
            return carry

        lax.fori_loop(0, TPW, body, 0)
        pltpu.sync_copy(r0_v, out_hbm.at[pl.ds(base, TPW)])

    return comb


_make_combine = functools.cache(_make_combine)


def _router_kernel(x_ref, rwt_ref, wv_ref, iv_ref, ps_ref):
    T = T_TOKENS
    xl = x_ref[...]
    logits = jnp.dot(xl.astype(jnp.bfloat16), rwt_ref[...].astype(jnp.bfloat16),
                     preferred_element_type=jnp.float32)
    lane = lax.broadcasted_iota(jnp.int32, (T, 128), 1)
    ml = jnp.where(lane < N_EXPERTS, logits, -jnp.inf)
    mx = jnp.max(ml, axis=1, keepdims=True)
    ex = jnp.exp(ml - mx)
    probs = ex / jnp.sum(ex, axis=1, keepdims=True)
    m0 = jnp.max(probs, axis=1, keepdims=True)
    i0 = jnp.min(jnp.where(probs == m0, lane, 128), axis=1, keepdims=True)
    pr1 = jnp.where(lane == i0, -1.0, probs)
    m1 = jnp.max(pr1, axis=1, keepdims=True)
    i1 = jnp.min(jnp.where(pr1 == m1, lane, 128), axis=1, keepdims=True)
    den = m0 + m1 + 1e-8
    wv_ref[...] = jnp.where(lane < 16, m0 / den,
                            jnp.where(lane < 32, m1 / den, 0.0))
    iv_ref[...] = jnp.where(lane == 0, i0, i1)
    ps_ref[...] = jnp.broadcast_to(jnp.sum(probs, axis=0, keepdims=True), (8, 128))


def _router(xf, router_w):
    T = T_TOKENS
    rwt = jnp.pad(router_w, ((0, 128 - N_EXPERTS), (0, 0))).T
    return pl.pallas_call(
        _router_kernel,
        out_shape=(
            jax.ShapeDtypeStruct((T, 128), jnp.float32),
            jax.ShapeDtypeStruct((T, 128), jnp.int32),
            jax.ShapeDtypeStruct((8, 128), jnp.float32),
        ),
    )(xf, rwt)


def _tree_sum(parts):
    while len(parts) > 1:
        nxt = [parts[i] + parts[i + 1] for i in range(0, len(parts) - 1, 2)]
        if len(parts) % 2:
            nxt.append(parts[-1])
        parts = nxt
    return parts[0]


def _ffn_kernel(counts_s, xp_ref, w1c0, w1c1, w1c2, w2c0, w2c1, w2c2,
                eo_ref, w1q, w2q):
    e = pl.program_id(0)
    m = pl.program_id(1)

    @pl.when(m == 0)
    def _quantize_weights():
        def ternarize(chunks, q_ref):
            parts = []
            for ch, rows, nsub in chunks:
                sz = rows // nsub
                for i in range(nsub):
                    parts.append(jnp.sum(jnp.abs(ch[0, pl.ds(i * sz, sz), :])))
            s = 1.0 / jnp.clip(_tree_sum(parts) / (D_MODEL * D_FF), 1e-5, None)
            d = 1.0 / s
            t = 0.5 * d
            off = 0
            for ch, rows, nsub in chunks:
                sz = rows // nsub
                for i in range(nsub):
                    w = ch[0, pl.ds(i * sz, sz), :]
                    q_ref[pl.ds(off + i * sz, sz), :] = jnp.where(
                        w > t, d, jnp.where(w < -t, -d, 0.0)).astype(jnp.bfloat16)
                off += rows

        ternarize([(w1c0, 320, 2), (w1c1, 320, 2), (w1c2, 128, 1)], w1q)
        ternarize([(w2c0, 1280, 4), (w2c1, 1280, 4), (w2c2, 512, 2)], w2q)

    @pl.when(m * TM < counts_s[e])
    def _compute():
        xt = xp_ref[0]
        ax = jnp.clip(jnp.max(jnp.abs(xt), axis=-1, keepdims=True), 1e-5, None)
        sx = 127.0 / ax
        rx = ax * (1.0 / 127.0)
        xq = jnp.clip(jnp.round(xt * sx), -128.0, 127.0) * rx
        h = jnp.dot(xq.astype(jnp.bfloat16), w1q[...],
                    preferred_element_type=jnp.float32)
        h = jax.nn.gelu(h)
        ah = jnp.clip(jnp.max(jnp.abs(h), axis=-1, keepdims=True), 1e-5, None)
        sh = 127.0 / ah
        rh = ah * (1.0 / 127.0)
        hq = jnp.clip(jnp.round(h * sh), -128.0, 127.0) * rh
        o = jnp.dot(hq.astype(jnp.bfloat16), w2q[...],
                    preferred_element_type=jnp.float32)
        eo_ref[0] = o


def _row_tile_map(e, m, c):
    last = jnp.maximum(lax.div(c[e] + TM - 1, TM) - 1, 0)
    return (e, jnp.minimum(m, last), 0)


def _chunk_map(adv, blk):
    def f(e, m, c):
        return (jnp.where(m > adv, jnp.minimum(e + 1, N_EXPERTS - 1), e), blk, 0)
    return f


def _ffn(xp, w1, w2, counts):
    grid = (N_EXPERTS, CAP // TM)
    return pl.pallas_call(
        _ffn_kernel,
        grid_spec=pltpu.PrefetchScalarGridSpec(
            num_scalar_prefetch=1,
            grid=grid,
            in_specs=[
                pl.BlockSpec((1, TM, D_MODEL), _row_tile_map),
                pl.BlockSpec((1, 320, D_FF), _chunk_map(0, 0)),
                pl.BlockSpec((1, 320, D_FF), _chunk_map(1, 1)),
                pl.BlockSpec((1, 128, D_FF), _chunk_map(2, 5)),
                pl.BlockSpec((1, 1280, D_MODEL), _chunk_map(0, 0)),
                pl.BlockSpec((1, 1280, D_MODEL), _chunk_map(1, 1)),
                pl.BlockSpec((1, 512, D_MODEL), _chunk_map(2, 5)),
            ],
            out_specs=pl.BlockSpec((1, TM, D_MODEL), _row_tile_map),
            scratch_shapes=[
                pltpu.VMEM((D_MODEL, D_FF), jnp.bfloat16),
                pltpu.VMEM((D_FF, D_MODEL), jnp.bfloat16),
            ],
        ),
        out_shape=jax.ShapeDtypeStruct((N_EXPERTS, CAP, D_MODEL), jnp.float32),
    )(counts, xp, w1, w1, w1, w2, w2, w2)


def kernel(x, router_w, w1, w2):
    B, T, D = x.shape
    xf = x.reshape(T, D)

    wvb, ivb, psb = _router(xf, router_w)
    idx_all = jnp.concatenate([ivb[:, 0], ivb[:, 1]])
    oh = jax.nn.one_hot(idx_all, N_EXPERTS, dtype=jnp.int32)
    rank = jnp.cumsum(oh, axis=0) - oh
    local = jnp.minimum(jnp.sum(rank * oh, axis=1), CAP - 1)
    dest = idx_all * CAP + local
    counts = jnp.sum(oh, axis=0).astype(jnp.int32)

    xp = _make_dispatch()(xf, dest)
    eo = _ffn(xp.reshape(N_EXPERTS, CAP, D_MODEL), w1, w2, counts)
    out = _make_combine()(eo.reshape(N_EXPERTS * CAP, D_MODEL),
                          dest[:T], dest[T:], wvb)

    f = counts.astype(jnp.float32) / (T * TOP_K)
    p = psb[0, :N_EXPERTS] / T
    aux_loss = N_EXPERTS * jnp.sum(f * p)
    return out.reshape(B, T, D), aux_loss
